# baseline (speedup 1.0000x reference)
import functools

import jax
import jax.numpy as jnp
from jax import lax
from jax.experimental import pallas as pl
from jax.experimental.pallas import tpu as pltpu
from jax.experimental.pallas import tpu_sc as plsc

VOCAB = 100000
EMBED_DIM = 64
BATCH = 4096
HIST = 50

NC, NS = 2, 16
NW = NC * NS
BLK = BATCH // NW
RING = 5

_MESH = plsc.VectorSubcoreMesh(core_axis_name="c", subcore_axis_name="s")


@functools.partial(
    pl.kernel,
    out_type=jax.ShapeDtypeStruct((HIST, EMBED_DIM // 8, NW, 8, BLK),
                                  jnp.float32),
    mesh=_MESH,
    compiler_params=pltpu.CompilerParams(use_tc_tiling_on_sc=False,
                                         needs_layout_passes=False),
    scratch_types=[
        pltpu.VMEM((HIST, BLK), jnp.int32),
        pltpu.VMEM((HIST, BLK), jnp.int32),
        [pltpu.VMEM((BLK, EMBED_DIM), jnp.float32)] * RING,
        [pltpu.VMEM((EMBED_DIM // 8, 8, BLK), jnp.float32)] * RING,
        pltpu.SemaphoreType.DMA,
        [pltpu.SemaphoreType.DMA] * RING,
        [pltpu.SemaphoreType.DMA] * RING,
    ],
)
def _gather_kernel(ids_hbm, remap_hbm, table_hbm, out_hbm,
                   ids_v, idx2_v, rows_v, tr_v, sem_remap, sems_g, sems_wb):
    wid = lax.axis_index("s") * NC + lax.axis_index("c")

    pltpu.sync_copy(ids_hbm.at[:, wid], ids_v)

    def fire_remap(h, carry):
        pltpu.async_copy(remap_hbm.at[ids_v.at[h]], idx2_v.at[h], sem_remap)
        return carry

    lax.fori_loop(0, HIST, fire_remap, 0)
    pltpu.make_async_copy(ids_hbm.at[:, wid], idx2_v, sem_remap).wait()

    lanes = lax.broadcasted_iota(jnp.int32, (16,), 0)
    row_sets = [k * 16 + lanes for k in range(BLK // 16)]

    def transpose_slot(b):
        src, dst = rows_v[b], tr_v[b]

        def per_te(te, carry):
            base_e = te * 8
            for ei in range(8):
                col = jnp.full((16,), base_e + ei, jnp.int32)
                for k in range(BLK // 16):
                    vals = plsc.load_gather(src, [row_sets[k], col])
                    dst[te, ei, pl.ds(k * 16, 16)] = vals
            return carry

        lax.fori_loop(0, EMBED_DIM // 8, per_te, 0)

    for b in range(RING):
        pltpu.async_copy(table_hbm.at[idx2_v.at[b]], rows_v[b], sems_g[b])

    def group(g, carry):
        for b in range(RING):
            h = g * RING + b
            pltpu.make_async_copy(table_hbm.at[idx2_v.at[0]], rows_v[b],
                                  sems_g[b]).wait()

            @pl.when(h >= RING)
            def _():
                pltpu.make_async_copy(tr_v[b], out_hbm.at[0, :, wid],
                                      sems_wb[b]).wait()

            transpose_slot(b)

            @pl.when(h + RING < HIST)
            def _():
                pltpu.async_copy(table_hbm.at[idx2_v.at[h + RING]],
                                 rows_v[b], sems_g[b])

            pltpu.async_copy(tr_v[b], out_hbm.at[h, :, wid], sems_wb[b])
        return carry

    lax.fori_loop(0, HIST // RING, group, 0)

    for b in range(RING):
        pltpu.make_async_copy(tr_v[b], out_hbm.at[0, :, wid],
                              sems_wb[b]).wait()


def kernel(input_ids, attention_mask, embedding_dict, input_ids2dict_ids):
    ids_t = input_ids.T.reshape(HIST, NW, BLK)
    out5 = _gather_kernel(ids_t, input_ids2dict_ids, embedding_dict)
    out = out5.transpose(2, 4, 0, 1, 3).reshape(BATCH, HIST, EMBED_DIM)
    return out, attention_mask
